# baseline (speedup 1.0000x reference)
import functools

import jax
import jax.numpy as jnp
from jax import lax
from jax.experimental import pallas as pl
from jax.experimental.pallas import tpu as pltpu
from jax.experimental.pallas import tpu_sc as plsc


def _sc_body(ar_hbm, emb_hbm, out_hbm, ar_v, idx_v, rows_v, sem):
    cid = lax.axis_index("c")
    sid = lax.axis_index("s")

    @pl.when((cid == 0) & (sid == 0))
    def _():
        pltpu.sync_copy(ar_hbm, ar_v)
        lane = lax.iota(jnp.int32, 16)
        arvec = ar_v[...]
        for k in range(2):
            acc = jnp.zeros((16,), jnp.int32)
            for j in range(16):
                bt = 16 * k + j
                b, t = bt // 4, bt % 4
                h = arvec[2 * b + 1]
                idx = (t // h) * 4 + (t % h)
                acc = jnp.where(lane == j, idx, acc)
            idx_v[pl.ds(16 * k, 16)] = acc
        pltpu.async_copy(emb_hbm.at[idx_v], rows_v, sem).wait()
        pltpu.sync_copy(rows_v, out_hbm)


def _sc_gather(ar16, emb2):
    mesh = plsc.VectorSubcoreMesh(core_axis_name="c", subcore_axis_name="s")
    fn = functools.partial(
        pl.kernel,
        mesh=mesh,
        out_type=jax.ShapeDtypeStruct((32, 1280), jnp.float32),
        scratch_types=[
            pltpu.VMEM((16,), jnp.int32),
            pltpu.VMEM((32,), jnp.int32),
            pltpu.VMEM((32, 1280), jnp.float32),
            pltpu.SemaphoreType.DMA,
        ],
    )(_sc_body)
    return fn(ar16, emb2)


def _tc_body(ar_ref, gate_ref, x_ref, tab_ref, o_ref):
    bi = pl.program_id(0)
    w = ar_ref[bi, 0]
    h = ar_ref[bi, 1]
    g = jnp.tanh(gate_ref[0])
    t = x_ref.shape[2]
    tile_id = jax.lax.broadcasted_iota(jnp.int32, (t, 1), 0)
    scale = jnp.where(tile_id < w * h, g, jnp.zeros_like(g))
    table = (tab_ref[...] * scale)[:, None]
    o_ref[...] = x_ref[...] + table


def kernel(x, ar, embedding, gate):
    b, t, n, w = x.shape
    xt = jnp.transpose(x, (0, 2, 1, 3))
    table = _sc_gather(ar.reshape(2 * b), embedding.reshape(t * t, w))
    table = table.reshape(b, t, w)
    tb = 480
    ntb = pl.cdiv(n, tb)

    def x_map(bi, ni, ar_ref, gate_ref):
        return (bi, ni, 0, 0)

    def tab_map(bi, ni, ar_ref, gate_ref):
        return (bi, 0, 0)

    grid_spec = pltpu.PrefetchScalarGridSpec(
        num_scalar_prefetch=2,
        grid=(b, ntb),
        in_specs=[
            pl.BlockSpec((1, tb, t, w), x_map),
            pl.BlockSpec((1, t, w), tab_map),
        ],
        out_specs=pl.BlockSpec((1, tb, t, w), x_map),
    )
    res = pl.pallas_call(
        _tc_body,
        grid_spec=grid_spec,
        out_shape=jax.ShapeDtypeStruct(xt.shape, x.dtype),
        compiler_params=pltpu.CompilerParams(
            dimension_semantics=("parallel", "arbitrary")),
    )(ar, gate, xt, table)
    return jnp.transpose(res, (0, 2, 1, 3))


# speedup vs baseline: 1.0137x; 1.0137x over previous
import functools

import jax
import jax.numpy as jnp
from jax import lax
from jax.experimental import pallas as pl
from jax.experimental.pallas import tpu as pltpu
from jax.experimental.pallas import tpu_sc as plsc


def _sc_body(ar_hbm, emb_hbm, out_hbm, ar_v, idx_v, rows_v, sem):
    cid = lax.axis_index("c")
    sid = lax.axis_index("s")

    @pl.when((cid == 0) & (sid == 0))
    def _():
        pltpu.sync_copy(ar_hbm, ar_v)
        lane = lax.iota(jnp.int32, 16)
        arvec = ar_v[...]
        for k in range(2):
            acc = jnp.zeros((16,), jnp.int32)
            for j in range(16):
                bt = 16 * k + j
                b, t = bt // 4, bt % 4
                h = arvec[2 * b + 1]
                idx = (t // h) * 4 + (t % h)
                acc = jnp.where(lane == j, idx, acc)
            idx_v[pl.ds(16 * k, 16)] = acc
        pltpu.async_copy(emb_hbm.at[idx_v], rows_v, sem).wait()
        pltpu.sync_copy(rows_v, out_hbm)


def _sc_gather(ar16, emb2):
    mesh = plsc.VectorSubcoreMesh(core_axis_name="c", subcore_axis_name="s", num_cores=1)
    fn = functools.partial(
        pl.kernel,
        mesh=mesh,
        out_type=jax.ShapeDtypeStruct((32, 1280), jnp.float32),
        scratch_types=[
            pltpu.VMEM((16,), jnp.int32),
            pltpu.VMEM((32,), jnp.int32),
            pltpu.VMEM((32, 1280), jnp.float32),
            pltpu.SemaphoreType.DMA,
        ],
    )(_sc_body)
    return fn(ar16, emb2)


def _tc_body(ar_ref, gate_ref, x_ref, tab_ref, o_ref):
    bi = pl.program_id(0)
    w = ar_ref[bi, 0]
    h = ar_ref[bi, 1]
    g = jnp.tanh(gate_ref[0])
    t = x_ref.shape[2]
    tile_id = jax.lax.broadcasted_iota(jnp.int32, (t, 1), 0)
    scale = jnp.where(tile_id < w * h, g, jnp.zeros_like(g))
    table = (tab_ref[...] * scale)[:, None]
    o_ref[...] = x_ref[...] + table


def kernel(x, ar, embedding, gate):
    b, t, n, w = x.shape
    xt = jnp.transpose(x, (0, 2, 1, 3))
    table = _sc_gather(ar.reshape(2 * b), embedding.reshape(t * t, w))
    table = table.reshape(b, t, w)
    tb = 480
    ntb = pl.cdiv(n, tb)

    def x_map(bi, ni, ar_ref, gate_ref):
        return (bi, ni, 0, 0)

    def tab_map(bi, ni, ar_ref, gate_ref):
        return (bi, 0, 0)

    grid_spec = pltpu.PrefetchScalarGridSpec(
        num_scalar_prefetch=2,
        grid=(b, ntb),
        in_specs=[
            pl.BlockSpec((1, tb, t, w), x_map),
            pl.BlockSpec((1, t, w), tab_map),
        ],
        out_specs=pl.BlockSpec((1, tb, t, w), x_map),
    )
    res = pl.pallas_call(
        _tc_body,
        grid_spec=grid_spec,
        out_shape=jax.ShapeDtypeStruct(xt.shape, x.dtype),
        compiler_params=pltpu.CompilerParams(
            dimension_semantics=("parallel", "arbitrary")),
    )(ar, gate, xt, table)
    return jnp.transpose(res, (0, 2, 1, 3))


# speedup vs baseline: 1.2275x; 1.2108x over previous
import jax
import jax.numpy as jnp
from jax.experimental import pallas as pl
from jax.experimental.pallas import tpu as pltpu


def _body(ar_ref, gate_ref, x_ref, emb_ref, o_ref):
    bi = pl.program_id(0)
    w = ar_ref[bi, 0]
    h = ar_ref[bi, 1]
    g = jnp.tanh(gate_ref[0])
    t = x_ref.shape[2]
    rows = []
    for ti in range(t):
        idx = (ti // h) * t + (ti % h)
        rows.append(emb_ref[idx, 0, :].reshape(1, -1))
    table = jnp.concatenate(rows, axis=0)
    tile_id = jax.lax.broadcasted_iota(jnp.int32, (t, 1), 0)
    scale = jnp.where(tile_id < w * h, g, jnp.zeros_like(g))
    table = (table * scale)[None, None]
    o_ref[...] = x_ref[...] + table


def kernel(x, ar, embedding, gate):
    b, t, n, w = x.shape
    xt = jnp.transpose(x, (0, 2, 1, 3))
    emb = embedding.reshape(t * t, 1, w)
    tb = 472
    ntb = pl.cdiv(n, tb)

    def x_map(bi, ni, ar_ref, gate_ref):
        return (bi, ni, 0, 0)

    def emb_map(bi, ni, ar_ref, gate_ref):
        return (0, 0, 0)

    grid_spec = pltpu.PrefetchScalarGridSpec(
        num_scalar_prefetch=2,
        grid=(b, ntb),
        in_specs=[
            pl.BlockSpec((1, tb, t, w), x_map),
            pl.BlockSpec((t * t, 1, w), emb_map),
        ],
        out_specs=pl.BlockSpec((1, tb, t, w), x_map),
    )
    res = pl.pallas_call(
        _body,
        grid_spec=grid_spec,
        out_shape=jax.ShapeDtypeStruct(xt.shape, x.dtype),
        compiler_params=pltpu.CompilerParams(
            dimension_semantics=("parallel", "arbitrary")),
    )(ar, gate, xt, emb)
    return jnp.transpose(res, (0, 2, 1, 3))


# speedup vs baseline: 1.2287x; 1.0010x over previous
import jax
import jax.numpy as jnp
from jax.experimental import pallas as pl
from jax.experimental.pallas import tpu as pltpu


def _body(ar_ref, gate_ref, x_ref, emb_ref, o_ref):
    bi = pl.program_id(0)
    w = ar_ref[bi, 0]
    h = ar_ref[bi, 1]
    g = jnp.tanh(gate_ref[0])
    t = x_ref.shape[2]
    rows = []
    for ti in range(t):
        idx = (ti // h) * t + (ti % h)
        rows.append(emb_ref[idx, 0, :].reshape(1, -1))
    table = jnp.concatenate(rows, axis=0)
    tile_id = jax.lax.broadcasted_iota(jnp.int32, (t, 1), 0)
    scale = jnp.where(tile_id < w * h, g, jnp.zeros_like(g))
    table = (table * scale)[None, None]
    o_ref[...] = x_ref[...] + table


def kernel(x, ar, embedding, gate):
    b, t, n, w = x.shape
    xt = jnp.transpose(x, (0, 2, 1, 3))
    emb = embedding.reshape(t * t, 1, w)
    tb = 480
    ntb = pl.cdiv(n, tb)

    def x_map(bi, ni, ar_ref, gate_ref):
        return (bi, ni, 0, 0)

    def emb_map(bi, ni, ar_ref, gate_ref):
        return (0, 0, 0)

    grid_spec = pltpu.PrefetchScalarGridSpec(
        num_scalar_prefetch=2,
        grid=(b, ntb),
        in_specs=[
            pl.BlockSpec((1, tb, t, w), x_map),
            pl.BlockSpec((t * t, 1, w), emb_map),
        ],
        out_specs=pl.BlockSpec((1, tb, t, w), x_map),
    )
    res = pl.pallas_call(
        _body,
        grid_spec=grid_spec,
        out_shape=jax.ShapeDtypeStruct(xt.shape, x.dtype),
        compiler_params=pltpu.CompilerParams(
            dimension_semantics=("parallel", "arbitrary")),
    )(ar, gate, xt, emb)
    return jnp.transpose(res, (0, 2, 1, 3))
